# Initial kernel scaffold; baseline (speedup 1.0000x reference)
import jax
import jax.numpy as jnp
from jax.experimental import pallas as pl


def kernel(V, K):
    raise NotImplementedError("write your pallas kernel here")


# speedup vs baseline: 4.0800x; 4.0800x over previous
import functools
import jax
import jax.numpy as jnp
from jax import lax
from jax.experimental import pallas as pl
from jax.experimental.pallas import tpu as pltpu
from jax.experimental.pallas import tpu_sc as plsc

NC, NS = 2, 16
NW = NC * NS
G = 80


def _gather_body(n_chunks, per_w, dv,
                 idxi_hbm, idxj_hbm, table_hbm, vi_hbm, vj_hbm,
                 idxi_v, idxj_v, rows_v, sem):
    wid = lax.axis_index("s") * NC + lax.axis_index("c")
    base = wid * per_w

    pltpu.sync_copy(idxi_hbm.at[wid], idxi_v)
    pltpu.sync_copy(idxj_hbm.at[wid], idxj_v)

    def step(c, _):
        pltpu.async_copy(table_hbm.at[idxi_v.at[c]], rows_v, sem).wait()
        pltpu.sync_copy(rows_v, vi_hbm.at[pl.ds(base + c * G, G)])
        pltpu.async_copy(table_hbm.at[idxj_v.at[c]], rows_v, sem).wait()
        pltpu.sync_copy(rows_v, vj_hbm.at[pl.ds(base + c * G, G)])
        return _

    lax.fori_loop(0, n_chunks, step, None)


def kernel(V, K):
    Z, N, Dv = V.shape
    Kk = K.shape[2]
    B = Z * N * Kk
    per_w = B // NW
    n_chunks = per_w // G

    table = V.reshape(Z * N, Dv)
    zoff = (jnp.arange(Z, dtype=jnp.int32) * N)[:, None, None]
    idxj = (K + zoff).reshape(NW, n_chunks, G)
    idxi = jnp.broadcast_to(
        jnp.arange(N, dtype=jnp.int32)[None, :, None] + zoff,
        (Z, N, Kk)).reshape(NW, n_chunks, G)

    mesh = plsc.VectorSubcoreMesh(
        core_axis_name="c", subcore_axis_name="s",
        num_cores=NC, num_subcores=NS)

    run = pl.kernel(
        functools.partial(_gather_body, n_chunks, per_w, Dv),
        out_type=(jax.ShapeDtypeStruct((B, Dv), jnp.float32),
                  jax.ShapeDtypeStruct((B, Dv), jnp.float32)),
        mesh=mesh,
        scratch_types=[
            pltpu.VMEM((n_chunks, G), jnp.int32),
            pltpu.VMEM((n_chunks, G), jnp.int32),
            pltpu.VMEM((G, Dv), jnp.float32),
            pltpu.SemaphoreType.DMA,
        ],
    )
    Vi_flat, Vj_flat = run(idxi, idxj, table)
    Vi = Vi_flat.reshape(Z, N, Kk, Dv)
    Vj = Vj_flat.reshape(Z, N, Kk, Dv)
    return (Vi, Vj)


# speedup vs baseline: 8.1916x; 2.0077x over previous
import functools
import jax
import jax.numpy as jnp
from jax import lax
from jax.experimental import pallas as pl
from jax.experimental.pallas import tpu as pltpu
from jax.experimental.pallas import tpu_sc as plsc

NC, NS = 2, 16
NW = NC * NS
G = 80
NBUF = 5


def _gather_body(n_chunks, per_w, dv,
                 idxi_hbm, idxj_hbm, table_hbm, vi_hbm, vj_hbm,
                 idxi_v, idxj_v, rows_v, *sems):
    gs = sems[:NBUF]
    ws = sems[NBUF:]
    wid = lax.axis_index("s") * NC + lax.axis_index("c")
    base = wid * per_w
    n_groups = n_chunks // NBUF

    pltpu.sync_copy(idxi_hbm.at[wid], idxi_v)
    pltpu.sync_copy(idxj_hbm.at[wid], idxj_v)

    def start_gather(idx_v, c, b):
        pltpu.async_copy(table_hbm.at[idx_v.at[c]], rows_v.at[b], gs[b])

    def wait_gather(b):
        pltpu.make_async_copy(table_hbm.at[idxi_v.at[0]], rows_v.at[b],
                              gs[b]).wait()

    def start_write(out_hbm, c, b):
        pltpu.async_copy(rows_v.at[b], out_hbm.at[pl.ds(base + c * G, G)],
                         ws[b])

    def wait_write(out_hbm, b):
        pltpu.make_async_copy(rows_v.at[b], out_hbm.at[pl.ds(base, G)],
                              ws[b]).wait()

    def run_output(idx_v, out_hbm):
        for b in range(NBUF):
            start_gather(idx_v, b, b)

        def group(g, _):
            for b in range(NBUF):
                wait_gather(b)
                start_write(out_hbm, g * NBUF + b, b)
            for b in range(NBUF):
                wait_write(out_hbm, b)
                start_gather(idx_v, (g + 1) * NBUF + b, b)
            return _

        lax.fori_loop(0, n_groups - 1, group, None)

        for b in range(NBUF):
            wait_gather(b)
            start_write(out_hbm, (n_groups - 1) * NBUF + b, b)
        for b in range(NBUF):
            wait_write(out_hbm, b)

    run_output(idxi_v, vi_hbm)
    run_output(idxj_v, vj_hbm)


def kernel(V, K):
    Z, N, Dv = V.shape
    Kk = K.shape[2]
    B = Z * N * Kk
    per_w = B // NW
    n_chunks = per_w // G

    table = V.reshape(Z * N, Dv)
    zoff = (jnp.arange(Z, dtype=jnp.int32) * N)[:, None, None]
    idxj = (K + zoff).reshape(NW, n_chunks, G)
    idxi = jnp.broadcast_to(
        jnp.arange(N, dtype=jnp.int32)[None, :, None] + zoff,
        (Z, N, Kk)).reshape(NW, n_chunks, G)

    mesh = plsc.VectorSubcoreMesh(
        core_axis_name="c", subcore_axis_name="s",
        num_cores=NC, num_subcores=NS)

    run = pl.kernel(
        functools.partial(_gather_body, n_chunks, per_w, Dv),
        out_type=(jax.ShapeDtypeStruct((B, Dv), jnp.float32),
                  jax.ShapeDtypeStruct((B, Dv), jnp.float32)),
        mesh=mesh,
        scratch_types=[
            pltpu.VMEM((n_chunks, G), jnp.int32),
            pltpu.VMEM((n_chunks, G), jnp.int32),
            pltpu.VMEM((NBUF, G, Dv), jnp.float32),
        ] + [pltpu.SemaphoreType.DMA] * (2 * NBUF),
    )
    Vi_flat, Vj_flat = run(idxi, idxj, table)
    Vi = Vi_flat.reshape(Z, N, Kk, Dv)
    Vj = Vj_flat.reshape(Z, N, Kk, Dv)
    return (Vi, Vj)


# speedup vs baseline: 14.9491x; 1.8249x over previous
import functools
import jax
import jax.numpy as jnp
from jax import lax
from jax.experimental import pallas as pl
from jax.experimental.pallas import tpu as pltpu
from jax.experimental.pallas import tpu_sc as plsc

NC, NS = 2, 16
NW = NC * NS
G = 80
NBUF = 5
RB = 400


def _gather_body(n_chunks, per_w, dv,
                 idxj_hbm, table_hbm, vj_hbm,
                 idxj_v, rows_v, *sems):
    gs = sems[:NBUF]
    ws = sems[NBUF:]
    wid = lax.axis_index("s") * NC + lax.axis_index("c")
    base = wid * per_w
    n_groups = n_chunks // NBUF

    pltpu.sync_copy(idxj_hbm.at[wid], idxj_v)

    def start_gather(c, b):
        pltpu.async_copy(table_hbm.at[idxj_v.at[c]], rows_v.at[b], gs[b])

    def wait_gather(b):
        pltpu.make_async_copy(table_hbm.at[idxj_v.at[0]], rows_v.at[b],
                              gs[b]).wait()

    def start_write(c, b):
        pltpu.async_copy(rows_v.at[b], vj_hbm.at[pl.ds(base + c * G, G)],
                         ws[b])

    def wait_write(b):
        pltpu.make_async_copy(rows_v.at[b], vj_hbm.at[pl.ds(base, G)],
                              ws[b]).wait()

    for b in range(NBUF):
        start_gather(b, b)

    def group(g, _):
        for b in range(NBUF):
            wait_gather(b)
            start_write(g * NBUF + b, b)
        for b in range(NBUF):
            wait_write(b)
            start_gather((g + 1) * NBUF + b, b)
        return _

    lax.fori_loop(0, n_groups - 1, group, None)

    for b in range(NBUF):
        wait_gather(b)
        start_write((n_groups - 1) * NBUF + b, b)
    for b in range(NBUF):
        wait_write(b)


def _broadcast_body(kk, dv, v_ref, out_ref):
    out_ref[...] = jnp.broadcast_to(v_ref[...][:, None, :],
                                    (v_ref.shape[0], kk, dv))


def kernel(V, K):
    Z, N, Dv = V.shape
    Kk = K.shape[2]
    B = Z * N * Kk
    per_w = B // NW
    n_chunks = per_w // G

    table = V.reshape(Z * N, Dv)
    zoff = (jnp.arange(Z, dtype=jnp.int32) * N)[:, None, None]
    idxj = (K + zoff).reshape(NW, n_chunks, G)

    vi_flat = pl.pallas_call(
        functools.partial(_broadcast_body, Kk, Dv),
        grid=(Z * N // RB,),
        in_specs=[pl.BlockSpec((RB, Dv), lambda i: (i, 0))],
        out_specs=pl.BlockSpec((RB, Kk, Dv), lambda i: (i, 0, 0)),
        out_shape=jax.ShapeDtypeStruct((Z * N, Kk, Dv), jnp.float32),
    )(table)
    Vi = vi_flat.reshape(Z, N, Kk, Dv)

    mesh = plsc.VectorSubcoreMesh(
        core_axis_name="c", subcore_axis_name="s",
        num_cores=NC, num_subcores=NS)

    run = pl.kernel(
        functools.partial(_gather_body, n_chunks, per_w, Dv),
        out_type=jax.ShapeDtypeStruct((B, Dv), jnp.float32),
        mesh=mesh,
        scratch_types=[
            pltpu.VMEM((n_chunks, G), jnp.int32),
            pltpu.VMEM((NBUF, G, Dv), jnp.float32),
        ] + [pltpu.SemaphoreType.DMA] * (2 * NBUF),
    )
    Vj = run(idxj, table).reshape(Z, N, Kk, Dv)
    return (Vi, Vj)


# speedup vs baseline: 15.1176x; 1.0113x over previous
import functools
import jax
import jax.numpy as jnp
from jax import lax
from jax.experimental import pallas as pl
from jax.experimental.pallas import tpu as pltpu
from jax.experimental.pallas import tpu_sc as plsc

NC, NS = 2, 16
NW = NC * NS
G = 80
NBUF = 10
LAG = 5
RB = 400


def _gather_body(n_chunks, per_w, dv,
                 idxj_hbm, table_hbm, vj_hbm,
                 idxj_v, rows_v, *sems):
    gs = sems[:NBUF]
    ws = sems[NBUF:]
    wid = lax.axis_index("s") * NC + lax.axis_index("c")
    base = wid * per_w
    n_groups = n_chunks // NBUF

    pltpu.sync_copy(idxj_hbm.at[wid], idxj_v)

    def start_gather(c, b):
        pltpu.async_copy(table_hbm.at[idxj_v.at[c]], rows_v.at[b], gs[b])

    def wait_gather(b):
        pltpu.make_async_copy(table_hbm.at[idxj_v.at[0]], rows_v.at[b],
                              gs[b]).wait()

    def start_write(c, b):
        pltpu.async_copy(rows_v.at[b], vj_hbm.at[pl.ds(base + c * G, G)],
                         ws[b])

    def wait_write(b):
        pltpu.make_async_copy(rows_v.at[b], vj_hbm.at[pl.ds(base, G)],
                              ws[b]).wait()

    assert n_chunks % NBUF == LAG and n_chunks >= NBUF + LAG

    for b in range(LAG):
        start_gather(b, b)
    for t in range(NBUF):
        bg = (LAG + t) % NBUF
        if LAG + t - NBUF >= 0:
            wait_write(bg)
        start_gather(LAG + t, bg)
        wait_gather(t)
        start_write(t, t)

    def block(s, _):
        c0 = s * NBUF
        for t in range(NBUF):
            bg = (LAG + t) % NBUF
            wait_write(bg)
            start_gather(c0 + LAG + t, bg)
            wait_gather(t)
            start_write(c0 + t, t)
        return _

    lax.fori_loop(1, (n_chunks - LAG) // NBUF, block, None)

    tail = n_chunks - LAG
    for r in range(LAG):
        wait_gather(r)
        start_write(tail + r, r)
    for b in range(NBUF):
        wait_write(b)


def _broadcast_body(kk, dv, v_ref, out_ref):
    out_ref[...] = jnp.broadcast_to(v_ref[...][:, None, :],
                                    (v_ref.shape[0], kk, dv))


def kernel(V, K):
    Z, N, Dv = V.shape
    Kk = K.shape[2]
    B = Z * N * Kk
    per_w = B // NW
    n_chunks = per_w // G

    table = V.reshape(Z * N, Dv)
    zoff = (jnp.arange(Z, dtype=jnp.int32) * N)[:, None, None]
    idxj = (K + zoff).reshape(NW, n_chunks, G)

    vi_flat = pl.pallas_call(
        functools.partial(_broadcast_body, Kk, Dv),
        grid=(Z * N // RB,),
        in_specs=[pl.BlockSpec((RB, Dv), lambda i: (i, 0))],
        out_specs=pl.BlockSpec((RB, Kk, Dv), lambda i: (i, 0, 0)),
        out_shape=jax.ShapeDtypeStruct((Z * N, Kk, Dv), jnp.float32),
    )(table)
    Vi = vi_flat.reshape(Z, N, Kk, Dv)

    mesh = plsc.VectorSubcoreMesh(
        core_axis_name="c", subcore_axis_name="s",
        num_cores=NC, num_subcores=NS)

    run = pl.kernel(
        functools.partial(_gather_body, n_chunks, per_w, Dv),
        out_type=jax.ShapeDtypeStruct((B, Dv), jnp.float32),
        mesh=mesh,
        scratch_types=[
            pltpu.VMEM((n_chunks, G), jnp.int32),
            pltpu.VMEM((NBUF, G, Dv), jnp.float32),
        ] + [pltpu.SemaphoreType.DMA] * (2 * NBUF),
    )
    Vj = run(idxj, table).reshape(Z, N, Kk, Dv)
    return (Vi, Vj)
